# baseline (speedup 1.0000x reference)
import functools

import jax
import jax.numpy as jnp
from jax import lax
from jax.experimental import pallas as pl
from jax.experimental.pallas import tpu as pltpu
from jax.experimental.pallas import tpu_sc as plsc

N = 10000
D = 128
DH = 64
DG = 80
HID = 256
E = 320000
NPAD = 10240
NC, NS, L = 2, 16, 16
NW = NC * NS
CHUNK = 80
NCH1 = E // NW // CHUNK
NCH2 = E // NS // CHUNK
RPT = NPAD // NS
ZR = 128

_SC_PARAMS = pltpu.CompilerParams(use_tc_tiling_on_sc=False)


def _mesh():
    return plsc.VectorSubcoreMesh(core_axis_name="c", subcore_axis_name="s")


@functools.partial(
    pl.kernel,
    out_type=jax.ShapeDtypeStruct((NC, 1, NPAD), jnp.float32),
    mesh=_mesh(),
    scratch_types=[
        pltpu.VMEM((NCH1, CHUNK), jnp.int32),
        pltpu.VMEM((CHUNK,), jnp.float32),
        pltpu.VMEM((RPT,), jnp.float32),
        pltpu.VMEM_SHARED((NPAD,), jnp.float32),
    ],
    compiler_params=_SC_PARAMS,
)
def _deg_kernel(dst_hbm, out_hbm, dstv, ones_v, zbuf, acc):
    cid = lax.axis_index("c")
    sid = lax.axis_index("s")
    w = cid * NS + sid
    zeros = jnp.zeros((L,), jnp.float32)
    ones = jnp.ones((L,), jnp.float32)
    for u in range(CHUNK // L):
        ones_v[pl.ds(u * L, L)] = ones

    @pl.loop(0, RPT // L)
    def _zz(u):
        zbuf[pl.ds(u * L, L)] = zeros

    pltpu.sync_copy(zbuf, acc.at[pl.ds(sid * RPT, RPT)])
    pltpu.sync_copy(dst_hbm.at[w], dstv)
    plsc.subcore_barrier()

    @pl.loop(0, NCH1)
    def _chunk(j):
        pltpu.sync_copy(ones_v, acc.at[dstv.at[j]], add=True)

    plsc.subcore_barrier()

    @pl.when(sid == 0)
    def _out():
        pltpu.sync_copy(acc, out_hbm.at[cid, 0])


def _make_pass(dw):
    @functools.partial(
        pl.kernel,
        out_type=jax.ShapeDtypeStruct((NC * NPAD, dw), jnp.float32),
        mesh=_mesh(),
        scratch_types=[
            pltpu.VMEM((NCH2, CHUNK), jnp.int32),
            pltpu.VMEM((NCH2, CHUNK), jnp.int32),
            pltpu.VMEM((4 * CHUNK, dw), jnp.float32),
            pltpu.VMEM((ZR, dw), jnp.float32),
            pltpu.VMEM_SHARED((NPAD, dw), jnp.float32),
            pltpu.SemaphoreType.DMA,
            pltpu.SemaphoreType.DMA,
        ],
        compiler_params=_SC_PARAMS,
    )
    def pass_kernel(table, srcp, dstp, agg_hbm, srcv, dstv, rows_v, zbuf,
                    acc, sem, ssem):
        cid = lax.axis_index("c")
        sid = lax.axis_index("s")
        zeros = jnp.zeros((L,), jnp.float32)
        tab = table.at[cid]

        @pl.loop(0, ZR)
        def _z(i):
            for u in range(dw // L):
                zbuf[i, pl.ds(u * L, L)] = zeros

        for k in range(RPT // ZR):
            pltpu.sync_copy(zbuf, acc.at[pl.ds(sid * RPT + k * ZR, ZR)])
        pltpu.sync_copy(srcp.at[sid], srcv)
        pltpu.sync_copy(dstp.at[sid], dstv)
        plsc.subcore_barrier()

        pltpu.async_copy(tab.at[srcv.at[0]], rows_v.at[pl.ds(0, CHUNK)],
                         sem)
        pltpu.async_copy(tab.at[srcv.at[1]],
                         rows_v.at[pl.ds(CHUNK, CHUNK)], sem)

        @pl.loop(0, NCH2)
        def _chunk(j):
            cur = lax.bitwise_and(j, 3) * CHUNK
            nxt = lax.bitwise_and(j + 2, 3) * CHUNK

            @pl.when(j >= 2)
            def _drain():
                pltpu.make_async_copy(rows_v.at[pl.ds(nxt, CHUNK)],
                                      acc.at[dstv.at[j - 2]], ssem).wait()

            @pl.when(j < NCH2 - 2)
            def _prefetch():
                pltpu.async_copy(tab.at[srcv.at[j + 2]],
                                 rows_v.at[pl.ds(nxt, CHUNK)], sem)

            pltpu.make_async_copy(tab.at[srcv.at[j]],
                                  rows_v.at[pl.ds(cur, CHUNK)], sem).wait()
            pltpu.async_copy(rows_v.at[pl.ds(cur, CHUNK)],
                             acc.at[dstv.at[j]], ssem, add=True)

        pltpu.make_async_copy(rows_v.at[pl.ds(2 * CHUNK, CHUNK)],
                              acc.at[dstv.at[NCH2 - 2]], ssem).wait()
        pltpu.make_async_copy(rows_v.at[pl.ds(3 * CHUNK, CHUNK)],
                              acc.at[dstv.at[NCH2 - 1]], ssem).wait()
        plsc.subcore_barrier()
        pltpu.sync_copy(acc.at[pl.ds(sid * RPT, RPT)],
                        agg_hbm.at[pl.ds(cid * NPAD + sid * RPT, RPT)])

    return pass_kernel


_pass1 = _make_pass(DG)
_pass2 = _make_pass(DH)


_GRID = 10
_BR = NPAD // _GRID


def _rsq(dp):
    return lax.rsqrt(dp[0] + dp[1] + 1.0)


def _scale_body(x, dp, o):
    h = pl.program_id(0) // _GRID
    rb = _rsq(dp)
    xb = x[...]
    xh = jnp.where(h == 0, xb[:, :DH], xb[:, DH:])
    lane = jax.lax.broadcasted_iota(jnp.int32, (_BR, DG - DH), 1)
    ghost = jnp.where(lane == 0, rb, 0.0)
    o[...] = jnp.concatenate([xh * rb, ghost], axis=1)[None]


_scale_call = pl.pallas_call(
    _scale_body,
    grid=(2 * _GRID,),
    in_specs=[
        pl.BlockSpec((_BR, D), lambda i: (i % _GRID, 0)),
        pl.BlockSpec((NC, _BR, 1), lambda i: (0, i % _GRID, 0)),
    ],
    out_specs=pl.BlockSpec((1, _BR, DG), lambda i: (i // _GRID, i % _GRID, 0)),
    out_shape=jax.ShapeDtypeStruct((NC, NPAD, DG), jnp.float32),
)


def _mid_body(ap, t1, dp, o):
    rb = _rsq(dp)
    rr = rb * rb
    o[...] = ((ap[:, :DH] + t1[0, :, :DH]) * rr)[None]


_mid_call = pl.pallas_call(
    _mid_body,
    grid=(2 * _GRID,),
    in_specs=[
        pl.BlockSpec((_BR, DG), lambda i: (i, 0)),
        pl.BlockSpec((1, _BR, DG), lambda i: (i // _GRID, i % _GRID, 0)),
        pl.BlockSpec((NC, _BR, 1), lambda i: (0, i % _GRID, 0)),
    ],
    out_specs=pl.BlockSpec((1, _BR, DH), lambda i: (i // _GRID, i % _GRID, 0)),
    out_shape=jax.ShapeDtypeStruct((NC, NPAD, DH), jnp.float32),
)


def _fin_body(apa, apb, t2a, t2b, ap1, dp, w1, w2, b1, b2, o):
    rb = _rsq(dp)
    ya = (apa[...] + t2a[0]) * rb
    yb = (apb[...] + t2b[0]) * rb
    y2 = jnp.concatenate([ya, yb], axis=1)
    wc = jnp.dot(w1[...], w2[...], preferred_element_type=jnp.float32)
    bv = jnp.dot(b1[...], w2[...], preferred_element_type=jnp.float32)
    c = (ap1[:, DH:DH + 1] + rb) * rb
    z = jnp.dot(y2, wc, preferred_element_type=jnp.float32) + c * bv + b2[...]
    o[...] = jax.nn.sigmoid(z)


_fin_call = pl.pallas_call(
    _fin_body,
    grid=(_GRID,),
    in_specs=[
        pl.BlockSpec((_BR, DH), lambda i: (i, 0)),
        pl.BlockSpec((_BR, DH), lambda i: (_GRID + i, 0)),
        pl.BlockSpec((1, _BR, DH), lambda i: (0, i, 0)),
        pl.BlockSpec((1, _BR, DH), lambda i: (1, i, 0)),
        pl.BlockSpec((_BR, DG), lambda i: (i, 0)),
        pl.BlockSpec((NC, _BR, 1), lambda i: (0, i, 0)),
        pl.BlockSpec((D, HID), lambda i: (0, 0)),
        pl.BlockSpec((HID, D), lambda i: (0, 0)),
        pl.BlockSpec((1, HID), lambda i: (0, 0)),
        pl.BlockSpec((1, D), lambda i: (0, 0)),
    ],
    out_specs=pl.BlockSpec((_BR, D), lambda i: (i, 0)),
    out_shape=jax.ShapeDtypeStruct((NPAD, D), jnp.float32),
)


def kernel(x, edge_index, W1, b1, W2, b2):
    src = edge_index[0].astype(jnp.int32)
    dst = edge_index[1].astype(jnp.int32)
    dst32 = dst.reshape(NW, NCH1, CHUNK)
    src16 = src.reshape(NS, NCH2, CHUNK)
    dstp = dst.reshape(NS, NCH2, CHUNK)
    xp = jnp.pad(x, ((0, NPAD - N), (0, 0)))

    degp = _deg_kernel(dst32)
    degp2 = degp.reshape(NC, NPAD, 1)

    t1 = _scale_call(xp, degp2)
    agg1 = _pass1(t1, src16, dstp)
    t2 = _mid_call(agg1, t1, degp2)
    agg2 = _pass2(t2, src16, dstp)

    out = _fin_call(agg2, agg2, t2, t2, agg1, degp2, W1, W2,
                    b1.reshape(1, HID), b2.reshape(1, D))
    return out[:N]


# speedup vs baseline: 1.0013x; 1.0013x over previous
import functools

import jax
import jax.numpy as jnp
from jax import lax
from jax.experimental import pallas as pl
from jax.experimental.pallas import tpu as pltpu
from jax.experimental.pallas import tpu_sc as plsc

N = 10000
D = 128
DH = 64
DG = 80
HID = 256
E = 320000
NPAD = 10240
NC, NS, L = 2, 16, 16
NW = NC * NS
CHUNK = 80
NCH1 = E // NW // CHUNK
NCH2 = E // NS // CHUNK
RPT = NPAD // NS
ZR = 128

_SC_PARAMS = pltpu.CompilerParams(use_tc_tiling_on_sc=False)


def _mesh():
    return plsc.VectorSubcoreMesh(core_axis_name="c", subcore_axis_name="s")


@functools.partial(
    pl.kernel,
    out_type=jax.ShapeDtypeStruct((NC, 1, NPAD), jnp.float32),
    mesh=_mesh(),
    scratch_types=[
        pltpu.VMEM((NCH1, CHUNK), jnp.int32),
        pltpu.VMEM((CHUNK,), jnp.float32),
        pltpu.VMEM((RPT,), jnp.float32),
        pltpu.VMEM_SHARED((NPAD,), jnp.float32),
    ],
    compiler_params=_SC_PARAMS,
)
def _deg_kernel(dst_hbm, out_hbm, dstv, ones_v, zbuf, acc):
    cid = lax.axis_index("c")
    sid = lax.axis_index("s")
    w = cid * NS + sid
    zeros = jnp.zeros((L,), jnp.float32)
    ones = jnp.ones((L,), jnp.float32)
    for u in range(CHUNK // L):
        ones_v[pl.ds(u * L, L)] = ones

    @pl.loop(0, RPT // L)
    def _zz(u):
        zbuf[pl.ds(u * L, L)] = zeros

    pltpu.sync_copy(zbuf, acc.at[pl.ds(sid * RPT, RPT)])
    pltpu.sync_copy(dst_hbm.at[w], dstv)
    plsc.subcore_barrier()

    @pl.loop(0, NCH1)
    def _chunk(j):
        pltpu.sync_copy(ones_v, acc.at[dstv.at[j]], add=True)

    plsc.subcore_barrier()

    @pl.when(sid == 0)
    def _out():
        pltpu.sync_copy(acc, out_hbm.at[cid, 0])


def _make_pass(dw):
    @functools.partial(
        pl.kernel,
        out_type=jax.ShapeDtypeStruct((NC * NPAD, dw), jnp.float32),
        mesh=_mesh(),
        scratch_types=[
            pltpu.VMEM((NCH2, CHUNK), jnp.int32),
            pltpu.VMEM((NCH2, CHUNK), jnp.int32),
            pltpu.VMEM((4 * CHUNK, dw), jnp.float32),
            pltpu.VMEM((ZR, dw), jnp.float32),
            pltpu.VMEM_SHARED((NPAD, dw), jnp.float32),
            pltpu.SemaphoreType.DMA,
            pltpu.SemaphoreType.DMA,
        ],
        compiler_params=_SC_PARAMS,
    )
    def pass_kernel(table, srcp, dstp, agg_hbm, srcv, dstv, rows_v, zbuf,
                    acc, sem, ssem):
        cid = lax.axis_index("c")
        sid = lax.axis_index("s")
        zeros = jnp.zeros((L,), jnp.float32)
        tab = table.at[cid]

        @pl.loop(0, ZR)
        def _z(i):
            for u in range(dw // L):
                zbuf[i, pl.ds(u * L, L)] = zeros

        for k in range(RPT // ZR):
            pltpu.sync_copy(zbuf, acc.at[pl.ds(sid * RPT + k * ZR, ZR)])
        pltpu.sync_copy(srcp.at[sid], srcv)
        pltpu.sync_copy(dstp.at[sid], dstv)
        plsc.subcore_barrier()

        pltpu.async_copy(tab.at[srcv.at[0]], rows_v.at[pl.ds(0, CHUNK)],
                         sem)
        pltpu.async_copy(tab.at[srcv.at[1]],
                         rows_v.at[pl.ds(CHUNK, CHUNK)], sem)

        @pl.loop(0, NCH2)
        def _chunk(j):
            cur = lax.bitwise_and(j, 3) * CHUNK
            nxt = lax.bitwise_and(j + 2, 3) * CHUNK

            @pl.when(j >= 2)
            def _drain():
                pltpu.make_async_copy(rows_v.at[pl.ds(nxt, CHUNK)],
                                      acc.at[dstv.at[j - 2]], ssem).wait()

            @pl.when(j < NCH2 - 2)
            def _prefetch():
                pltpu.async_copy(tab.at[srcv.at[j + 2]],
                                 rows_v.at[pl.ds(nxt, CHUNK)], sem)

            pltpu.make_async_copy(tab.at[srcv.at[j]],
                                  rows_v.at[pl.ds(cur, CHUNK)], sem).wait()
            pltpu.async_copy(rows_v.at[pl.ds(cur, CHUNK)],
                             acc.at[dstv.at[j]], ssem, add=True)

        pltpu.make_async_copy(rows_v.at[pl.ds(2 * CHUNK, CHUNK)],
                              acc.at[dstv.at[NCH2 - 2]], ssem).wait()
        pltpu.make_async_copy(rows_v.at[pl.ds(3 * CHUNK, CHUNK)],
                              acc.at[dstv.at[NCH2 - 1]], ssem).wait()
        plsc.subcore_barrier()
        pltpu.sync_copy(acc.at[pl.ds(sid * RPT, RPT)],
                        agg_hbm.at[pl.ds(cid * NPAD + sid * RPT, RPT)])

    return pass_kernel


_pass1 = _make_pass(DG)
_pass2 = _make_pass(DH)


_GRID = 10
_BR = NPAD // _GRID


def _rsq(dp):
    return lax.rsqrt(dp[0] + dp[1] + 1.0)


def _scale_body(x, dp, o):
    h = pl.program_id(0) // _GRID
    rb = _rsq(dp)
    xb = x[...]
    xh = jnp.where(h == 0, xb[:, :DH], xb[:, DH:])
    lane = jax.lax.broadcasted_iota(jnp.int32, (_BR, DG - DH), 1)
    ghost = jnp.where(lane == 0, rb, 0.0)
    o[...] = jnp.concatenate([xh * rb, ghost], axis=1)


_scale_call = pl.pallas_call(
    _scale_body,
    grid=(2 * _GRID,),
    in_specs=[
        pl.BlockSpec((_BR, D), lambda i: (i % _GRID, 0)),
        pl.BlockSpec((NC, _BR, 1), lambda i: (0, i % _GRID, 0)),
    ],
    out_specs=pl.BlockSpec((_BR, DG), lambda i: (i, 0)),
    out_shape=jax.ShapeDtypeStruct((NC * NPAD, DG), jnp.float32),
)


def _mid_body(ap, t1, dp, o):
    rb = _rsq(dp)
    rr = rb * rb
    o[...] = (ap[:, :DH] + t1[:, :DH]) * rr


_mid_call = pl.pallas_call(
    _mid_body,
    grid=(2 * _GRID,),
    in_specs=[
        pl.BlockSpec((_BR, DG), lambda i: (i, 0)),
        pl.BlockSpec((_BR, DG), lambda i: (i, 0)),
        pl.BlockSpec((NC, _BR, 1), lambda i: (0, i % _GRID, 0)),
    ],
    out_specs=pl.BlockSpec((_BR, DH), lambda i: (i, 0)),
    out_shape=jax.ShapeDtypeStruct((NC * NPAD, DH), jnp.float32),
)


def _fin_body(apa, apb, t2a, t2b, ap1, dp, w1, w2, b1, b2, o):
    rb = _rsq(dp)
    ya = (apa[...] + t2a[...]) * rb
    yb = (apb[...] + t2b[...]) * rb
    y2 = jnp.concatenate([ya, yb], axis=1)
    wc = jnp.dot(w1[...], w2[...], preferred_element_type=jnp.float32)
    bv = jnp.dot(b1[...], w2[...], preferred_element_type=jnp.float32)
    c = (ap1[:, DH:DH + 1] + rb) * rb
    z = jnp.dot(y2, wc, preferred_element_type=jnp.float32) + c * bv + b2[...]
    o[...] = jax.nn.sigmoid(z)


_fin_call = pl.pallas_call(
    _fin_body,
    grid=(_GRID,),
    in_specs=[
        pl.BlockSpec((_BR, DH), lambda i: (i, 0)),
        pl.BlockSpec((_BR, DH), lambda i: (_GRID + i, 0)),
        pl.BlockSpec((_BR, DH), lambda i: (i, 0)),
        pl.BlockSpec((_BR, DH), lambda i: (_GRID + i, 0)),
        pl.BlockSpec((_BR, DG), lambda i: (i, 0)),
        pl.BlockSpec((NC, _BR, 1), lambda i: (0, i, 0)),
        pl.BlockSpec((D, HID), lambda i: (0, 0)),
        pl.BlockSpec((HID, D), lambda i: (0, 0)),
        pl.BlockSpec((1, HID), lambda i: (0, 0)),
        pl.BlockSpec((1, D), lambda i: (0, 0)),
    ],
    out_specs=pl.BlockSpec((_BR, D), lambda i: (i, 0)),
    out_shape=jax.ShapeDtypeStruct((NPAD, D), jnp.float32),
)


def kernel(x, edge_index, W1, b1, W2, b2):
    src = edge_index[0].astype(jnp.int32)
    dst = edge_index[1].astype(jnp.int32)
    dst32 = dst.reshape(NW, NCH1, CHUNK)
    src16 = src.reshape(NS, NCH2, CHUNK)
    dstp = dst.reshape(NS, NCH2, CHUNK)
    xp = jnp.pad(x, ((0, NPAD - N), (0, 0)))

    degp = _deg_kernel(dst32)
    degp2 = degp.reshape(NC, NPAD, 1)

    t1 = _scale_call(xp, degp2)
    agg1 = _pass1(t1.reshape(NC, NPAD, DG), src16, dstp)
    t2 = _mid_call(agg1, t1, degp2)
    agg2 = _pass2(t2.reshape(NC, NPAD, DH), src16, dstp)

    out = _fin_call(agg2, agg2, t2, t2, agg1, degp2, W1, W2,
                    b1.reshape(1, HID), b2.reshape(1, D))
    return out[:N]


# speedup vs baseline: 1.0040x; 1.0028x over previous
import functools

import jax
import jax.numpy as jnp
from jax import lax
from jax.experimental import pallas as pl
from jax.experimental.pallas import tpu as pltpu
from jax.experimental.pallas import tpu_sc as plsc

N = 10000
D = 128
DH = 64
DG = 80
HID = 256
E = 320000
NPAD = 10240
NC, NS, L = 2, 16, 16
NW = NC * NS
CHUNK = 80
NCH1 = E // NW // CHUNK
NCH2 = E // NS // CHUNK
RPT = NPAD // NS
ZR = 128

_SC_PARAMS = pltpu.CompilerParams(use_tc_tiling_on_sc=False)


def _mesh():
    return plsc.VectorSubcoreMesh(core_axis_name="c", subcore_axis_name="s")


@functools.partial(
    pl.kernel,
    out_type=jax.ShapeDtypeStruct((NC, 1, NPAD), jnp.float32),
    mesh=_mesh(),
    scratch_types=[
        pltpu.VMEM((NCH1, CHUNK), jnp.int32),
        pltpu.VMEM((CHUNK,), jnp.float32),
        pltpu.VMEM((RPT,), jnp.float32),
        pltpu.VMEM_SHARED((NPAD,), jnp.float32),
    ],
    compiler_params=_SC_PARAMS,
)
def _deg_kernel(dst_hbm, out_hbm, dstv, ones_v, zbuf, acc):
    cid = lax.axis_index("c")
    sid = lax.axis_index("s")
    w = cid * NS + sid
    zeros = jnp.zeros((L,), jnp.float32)
    ones = jnp.ones((L,), jnp.float32)
    for u in range(CHUNK // L):
        ones_v[pl.ds(u * L, L)] = ones

    @pl.loop(0, RPT // L)
    def _zz(u):
        zbuf[pl.ds(u * L, L)] = zeros

    pltpu.sync_copy(zbuf, acc.at[pl.ds(sid * RPT, RPT)])
    pltpu.sync_copy(dst_hbm.at[w], dstv)
    plsc.subcore_barrier()

    @pl.loop(0, NCH1)
    def _chunk(j):
        pltpu.sync_copy(ones_v, acc.at[dstv.at[j]], add=True)

    plsc.subcore_barrier()

    @pl.when(sid == 0)
    def _out():
        pltpu.sync_copy(acc, out_hbm.at[cid, 0])


def _make_pass(dw):
    @functools.partial(
        pl.kernel,
        out_type=jax.ShapeDtypeStruct((NC * NPAD, dw), jnp.float32),
        mesh=_mesh(),
        scratch_types=[
            pltpu.VMEM((NCH2, CHUNK), jnp.int32),
            pltpu.VMEM((NCH2, CHUNK), jnp.int32),
            pltpu.VMEM((4 * CHUNK, dw), jnp.float32),
            pltpu.VMEM((ZR, dw), jnp.float32),
            pltpu.VMEM_SHARED((NPAD, dw), jnp.float32),
            pltpu.SemaphoreType.DMA,
            pltpu.SemaphoreType.DMA,
        ],
        compiler_params=_SC_PARAMS,
    )
    def pass_kernel(table, srcp, dstp, agg_hbm, srcv, dstv, rows_v, zbuf,
                    acc, sem, ssem):
        cid = lax.axis_index("c")
        sid = lax.axis_index("s")
        zeros = jnp.zeros((L,), jnp.float32)
        tab = table

        @pl.loop(0, ZR)
        def _z(i):
            for u in range(dw // L):
                zbuf[i, pl.ds(u * L, L)] = zeros

        for k in range(RPT // ZR):
            pltpu.sync_copy(zbuf, acc.at[pl.ds(sid * RPT + k * ZR, ZR)])
        pltpu.sync_copy(srcp.at[cid, sid], srcv)
        pltpu.sync_copy(dstp.at[sid], dstv)
        plsc.subcore_barrier()

        pltpu.async_copy(tab.at[srcv.at[0]], rows_v.at[pl.ds(0, CHUNK)],
                         sem)
        pltpu.async_copy(tab.at[srcv.at[1]],
                         rows_v.at[pl.ds(CHUNK, CHUNK)], sem)

        @pl.loop(0, NCH2)
        def _chunk(j):
            cur = lax.bitwise_and(j, 3) * CHUNK
            nxt = lax.bitwise_and(j + 2, 3) * CHUNK

            @pl.when(j >= 2)
            def _drain():
                pltpu.make_async_copy(rows_v.at[pl.ds(nxt, CHUNK)],
                                      acc.at[dstv.at[j - 2]], ssem).wait()

            @pl.when(j < NCH2 - 2)
            def _prefetch():
                pltpu.async_copy(tab.at[srcv.at[j + 2]],
                                 rows_v.at[pl.ds(nxt, CHUNK)], sem)

            pltpu.make_async_copy(tab.at[srcv.at[j]],
                                  rows_v.at[pl.ds(cur, CHUNK)], sem).wait()
            pltpu.async_copy(rows_v.at[pl.ds(cur, CHUNK)],
                             acc.at[dstv.at[j]], ssem, add=True)

        pltpu.make_async_copy(rows_v.at[pl.ds(2 * CHUNK, CHUNK)],
                              acc.at[dstv.at[NCH2 - 2]], ssem).wait()
        pltpu.make_async_copy(rows_v.at[pl.ds(3 * CHUNK, CHUNK)],
                              acc.at[dstv.at[NCH2 - 1]], ssem).wait()
        plsc.subcore_barrier()
        pltpu.sync_copy(acc.at[pl.ds(sid * RPT, RPT)],
                        agg_hbm.at[pl.ds(cid * NPAD + sid * RPT, RPT)])

    return pass_kernel


_pass1 = _make_pass(DG)
_pass2 = _make_pass(DH)


_GRID = 10
_BR = NPAD // _GRID


def _rsq(dp):
    return lax.rsqrt(dp[0] + dp[1] + 1.0)


def _scale_body(x, dp, o):
    h = pl.program_id(0) // _GRID
    rb = _rsq(dp)
    xb = x[...]
    xh = jnp.where(h == 0, xb[:, :DH], xb[:, DH:])
    lane = jax.lax.broadcasted_iota(jnp.int32, (_BR, DG - DH), 1)
    ghost = jnp.where(lane == 0, rb, 0.0)
    o[...] = jnp.concatenate([xh * rb, ghost], axis=1)


_scale_call = pl.pallas_call(
    _scale_body,
    grid=(2 * _GRID,),
    in_specs=[
        pl.BlockSpec((_BR, D), lambda i: (i % _GRID, 0)),
        pl.BlockSpec((NC, _BR, 1), lambda i: (0, i % _GRID, 0)),
    ],
    out_specs=pl.BlockSpec((_BR, DG), lambda i: (i, 0)),
    out_shape=jax.ShapeDtypeStruct((NC * NPAD, DG), jnp.float32),
)


def _mid_body(ap, t1, dp, o):
    rb = _rsq(dp)
    rr = rb * rb
    o[...] = (ap[:, :DH] + t1[:, :DH]) * rr


_mid_call = pl.pallas_call(
    _mid_body,
    grid=(2 * _GRID,),
    in_specs=[
        pl.BlockSpec((_BR, DG), lambda i: (i, 0)),
        pl.BlockSpec((_BR, DG), lambda i: (i, 0)),
        pl.BlockSpec((NC, _BR, 1), lambda i: (0, i % _GRID, 0)),
    ],
    out_specs=pl.BlockSpec((_BR, DH), lambda i: (i, 0)),
    out_shape=jax.ShapeDtypeStruct((NC * NPAD, DH), jnp.float32),
)


def _fin_body(apa, apb, t2a, t2b, ap1, dp, w1, w2, b1, b2, o):
    rb = _rsq(dp)
    ya = (apa[...] + t2a[...]) * rb
    yb = (apb[...] + t2b[...]) * rb
    y2 = jnp.concatenate([ya, yb], axis=1)
    wc = jnp.dot(w1[...], w2[...], preferred_element_type=jnp.float32)
    bv = jnp.dot(b1[...], w2[...], preferred_element_type=jnp.float32)
    c = (ap1[:, DH:DH + 1] + rb) * rb
    z = jnp.dot(y2, wc, preferred_element_type=jnp.float32) + c * bv + b2[...]
    o[...] = jax.nn.sigmoid(z)


_fin_call = pl.pallas_call(
    _fin_body,
    grid=(_GRID,),
    in_specs=[
        pl.BlockSpec((_BR, DH), lambda i: (i, 0)),
        pl.BlockSpec((_BR, DH), lambda i: (_GRID + i, 0)),
        pl.BlockSpec((_BR, DH), lambda i: (i, 0)),
        pl.BlockSpec((_BR, DH), lambda i: (_GRID + i, 0)),
        pl.BlockSpec((_BR, DG), lambda i: (i, 0)),
        pl.BlockSpec((NC, _BR, 1), lambda i: (0, i, 0)),
        pl.BlockSpec((D, HID), lambda i: (0, 0)),
        pl.BlockSpec((HID, D), lambda i: (0, 0)),
        pl.BlockSpec((1, HID), lambda i: (0, 0)),
        pl.BlockSpec((1, D), lambda i: (0, 0)),
    ],
    out_specs=pl.BlockSpec((_BR, D), lambda i: (i, 0)),
    out_shape=jax.ShapeDtypeStruct((NPAD, D), jnp.float32),
)


def kernel(x, edge_index, W1, b1, W2, b2):
    src = edge_index[0].astype(jnp.int32)
    dst = edge_index[1].astype(jnp.int32)
    dst32 = dst.reshape(NW, NCH1, CHUNK)
    src16 = src.reshape(NS, NCH2, CHUNK)
    srcp = jnp.stack([src16, src16 + NPAD])
    dstp = dst.reshape(NS, NCH2, CHUNK)
    xp = jnp.pad(x, ((0, NPAD - N), (0, 0)))

    degp = _deg_kernel(dst32)
    degp2 = degp.reshape(NC, NPAD, 1)

    t1 = _scale_call(xp, degp2)
    agg1 = _pass1(t1, srcp, dstp)
    t2 = _mid_call(agg1, t1, degp2)
    agg2 = _pass2(t2, srcp, dstp)

    out = _fin_call(agg2, agg2, t2, t2, agg1, degp2, W1, W2,
                    b1.reshape(1, HID), b2.reshape(1, D))
    return out[:N]


# speedup vs baseline: 1.0306x; 1.0264x over previous
import functools

import jax
import jax.numpy as jnp
from jax import lax
from jax.experimental import pallas as pl
from jax.experimental.pallas import tpu as pltpu
from jax.experimental.pallas import tpu_sc as plsc

N = 10000
D = 128
DH = 64
DG = 80
HID = 256
E = 320000
NPAD = 10240
NC, NS, L = 2, 16, 16
NW = NC * NS
CHUNK = 80
NCH1 = E // NW // CHUNK
NCH2 = E // NS // CHUNK
RPT = NPAD // NS
ZR = 128

_SC_PARAMS = pltpu.CompilerParams(use_tc_tiling_on_sc=False)


def _mesh():
    return plsc.VectorSubcoreMesh(core_axis_name="c", subcore_axis_name="s")


@functools.partial(
    pl.kernel,
    out_type=jax.ShapeDtypeStruct((NC, 1, NPAD), jnp.float32),
    mesh=_mesh(),
    scratch_types=[
        pltpu.VMEM((NCH1, CHUNK), jnp.int32),
        pltpu.VMEM((CHUNK,), jnp.float32),
        pltpu.VMEM((RPT,), jnp.float32),
        pltpu.VMEM_SHARED((NPAD,), jnp.float32),
    ],
    compiler_params=_SC_PARAMS,
)
def _deg_kernel(dst_hbm, out_hbm, dstv, ones_v, zbuf, acc):
    cid = lax.axis_index("c")
    sid = lax.axis_index("s")
    w = cid * NS + sid
    zeros = jnp.zeros((L,), jnp.float32)
    ones = jnp.ones((L,), jnp.float32)
    for u in range(CHUNK // L):
        ones_v[pl.ds(u * L, L)] = ones

    @pl.loop(0, RPT // L)
    def _zz(u):
        zbuf[pl.ds(u * L, L)] = zeros

    pltpu.sync_copy(zbuf, acc.at[pl.ds(sid * RPT, RPT)])
    pltpu.sync_copy(dst_hbm.at[w], dstv)
    plsc.subcore_barrier()

    @pl.loop(0, NCH1)
    def _chunk(j):
        pltpu.sync_copy(ones_v, acc.at[dstv.at[j]], add=True)

    plsc.subcore_barrier()

    @pl.when(sid == 0)
    def _out():
        pltpu.sync_copy(acc, out_hbm.at[cid, 0])


def _make_pass(dw):
    @functools.partial(
        pl.kernel,
        out_type=jax.ShapeDtypeStruct((NC * NPAD, dw), jnp.float32),
        mesh=_mesh(),
        scratch_types=[
            pltpu.VMEM((NCH2, CHUNK), jnp.int32),
            pltpu.VMEM((NCH2, CHUNK), jnp.int32),
            pltpu.VMEM((4 * CHUNK, dw), jnp.float32),
            pltpu.VMEM((ZR, dw), jnp.float32),
            pltpu.VMEM_SHARED((NPAD, dw), jnp.float32),
            pltpu.SemaphoreType.DMA,
            pltpu.SemaphoreType.DMA,
        ],
        compiler_params=_SC_PARAMS,
    )
    def pass_kernel(table, srcp, dstp, agg_hbm, srcv, dstv, rows_v, zbuf,
                    acc, sem, ssem):
        cid = lax.axis_index("c")
        sid = lax.axis_index("s")
        zeros = jnp.zeros((L,), jnp.float32)
        tab = table

        @pl.loop(0, ZR)
        def _z(i):
            for u in range(dw // L):
                zbuf[i, pl.ds(u * L, L)] = zeros

        for k in range(RPT // ZR):
            pltpu.sync_copy(zbuf, acc.at[pl.ds(sid * RPT + k * ZR, ZR)])
        pltpu.sync_copy(srcp.at[cid, sid], srcv)
        pltpu.sync_copy(dstp.at[sid], dstv)
        plsc.subcore_barrier()

        pltpu.async_copy(tab.at[srcv.at[0]], rows_v.at[pl.ds(0, CHUNK)],
                         sem)
        pltpu.async_copy(tab.at[srcv.at[1]],
                         rows_v.at[pl.ds(CHUNK, CHUNK)], sem)

        @pl.loop(0, NCH2)
        def _chunk(j):
            cur = lax.bitwise_and(j, 3) * CHUNK
            nxt = lax.bitwise_and(j + 2, 3) * CHUNK

            @pl.when(j >= 2)
            def _drain():
                pltpu.make_async_copy(rows_v.at[pl.ds(nxt, CHUNK)],
                                      acc.at[dstv.at[j - 2]], ssem).wait()

            @pl.when(j < NCH2 - 2)
            def _prefetch():
                pltpu.async_copy(tab.at[srcv.at[j + 2]],
                                 rows_v.at[pl.ds(nxt, CHUNK)], sem)

            pltpu.make_async_copy(tab.at[srcv.at[j]],
                                  rows_v.at[pl.ds(cur, CHUNK)], sem).wait()
            pltpu.async_copy(rows_v.at[pl.ds(cur, CHUNK)],
                             acc.at[dstv.at[j]], ssem, add=True)

        pltpu.make_async_copy(rows_v.at[pl.ds(2 * CHUNK, CHUNK)],
                              acc.at[dstv.at[NCH2 - 2]], ssem).wait()
        pltpu.make_async_copy(rows_v.at[pl.ds(3 * CHUNK, CHUNK)],
                              acc.at[dstv.at[NCH2 - 1]], ssem).wait()
        plsc.subcore_barrier()
        pltpu.sync_copy(acc.at[pl.ds(sid * RPT, RPT)],
                        agg_hbm.at[pl.ds(cid * NPAD + sid * RPT, RPT)])

    return pass_kernel


_pass1 = _make_pass(DG)
_pass2 = _make_pass(DH)


def _rsqrt_body(dp, r):
    r[...] = lax.rsqrt(dp[0, 0] + dp[1, 0] + 1.0)


_rsqrt_call = pl.pallas_call(
    _rsqrt_body,
    out_shape=jax.ShapeDtypeStruct((NPAD // D, D), jnp.float32),
)

_GRID = 10
_BR = NPAD // _GRID


def _scale_body(x, r, o):
    h = pl.program_id(0) // _GRID
    rb = r[...]
    xb = x[...]
    xh = jnp.where(h == 0, xb[:, :DH], xb[:, DH:])
    lane = jax.lax.broadcasted_iota(jnp.int32, (_BR, DG - DH), 1)
    ghost = jnp.where(lane == 0, rb, 0.0)
    o[...] = jnp.concatenate([xh * rb, ghost], axis=1)


_scale_call = pl.pallas_call(
    _scale_body,
    grid=(2 * _GRID,),
    in_specs=[
        pl.BlockSpec((_BR, D), lambda i: (i % _GRID, 0)),
        pl.BlockSpec((_BR, 1), lambda i: (i % _GRID, 0)),
    ],
    out_specs=pl.BlockSpec((_BR, DG), lambda i: (i, 0)),
    out_shape=jax.ShapeDtypeStruct((NC * NPAD, DG), jnp.float32),
)


def _mid_body(ap, t1, r, o):
    rb = r[...]
    rr = rb * rb
    o[...] = (ap[:, :DH] + t1[:, :DH]) * rr


_mid_call = pl.pallas_call(
    _mid_body,
    grid=(2 * _GRID,),
    in_specs=[
        pl.BlockSpec((_BR, DG), lambda i: (i, 0)),
        pl.BlockSpec((_BR, DG), lambda i: (i, 0)),
        pl.BlockSpec((_BR, 1), lambda i: (i % _GRID, 0)),
    ],
    out_specs=pl.BlockSpec((_BR, DH), lambda i: (i, 0)),
    out_shape=jax.ShapeDtypeStruct((NC * NPAD, DH), jnp.float32),
)


def _fin_body(apa, apb, t2a, t2b, ap1, r, w1, w2, b1, b2, o):
    rb = r[...]
    ya = (apa[...] + t2a[...]) * rb
    yb = (apb[...] + t2b[...]) * rb
    y2 = jnp.concatenate([ya, yb], axis=1)
    wc = jnp.dot(w1[...], w2[...], preferred_element_type=jnp.float32)
    bv = jnp.dot(b1[...], w2[...], preferred_element_type=jnp.float32)
    c = (ap1[:, DH:DH + 1] + rb) * rb
    z = jnp.dot(y2, wc, preferred_element_type=jnp.float32) + c * bv + b2[...]
    o[...] = jax.nn.sigmoid(z)


_fin_call = pl.pallas_call(
    _fin_body,
    grid=(_GRID,),
    in_specs=[
        pl.BlockSpec((_BR, DH), lambda i: (i, 0)),
        pl.BlockSpec((_BR, DH), lambda i: (_GRID + i, 0)),
        pl.BlockSpec((_BR, DH), lambda i: (i, 0)),
        pl.BlockSpec((_BR, DH), lambda i: (_GRID + i, 0)),
        pl.BlockSpec((_BR, DG), lambda i: (i, 0)),
        pl.BlockSpec((_BR, 1), lambda i: (i, 0)),
        pl.BlockSpec((D, HID), lambda i: (0, 0)),
        pl.BlockSpec((HID, D), lambda i: (0, 0)),
        pl.BlockSpec((1, HID), lambda i: (0, 0)),
        pl.BlockSpec((1, D), lambda i: (0, 0)),
    ],
    out_specs=pl.BlockSpec((_BR, D), lambda i: (i, 0)),
    out_shape=jax.ShapeDtypeStruct((NPAD, D), jnp.float32),
)


def kernel(x, edge_index, W1, b1, W2, b2):
    src = edge_index[0].astype(jnp.int32)
    dst = edge_index[1].astype(jnp.int32)
    dst32 = dst.reshape(NW, NCH1, CHUNK)
    src16 = src.reshape(NS, NCH2, CHUNK)
    srcp = jnp.stack([src16, src16 + NPAD])
    dstp = dst.reshape(NS, NCH2, CHUNK)
    xp = jnp.pad(x, ((0, NPAD - N), (0, 0)))

    degp = _deg_kernel(dst32)
    r = _rsqrt_call(degp.reshape(NC, 1, NPAD // D, D))
    r2d = r.reshape(NPAD, 1)

    t1 = _scale_call(xp, r2d)
    agg1 = _pass1(t1, srcp, dstp)
    t2 = _mid_call(agg1, t1, r2d)
    agg2 = _pass2(t2, srcp, dstp)

    out = _fin_call(agg2, agg2, t2, t2, agg1, r2d, W1, W2,
                    b1.reshape(1, HID), b2.reshape(1, D))
    return out[:N]


# speedup vs baseline: 1.0323x; 1.0017x over previous
import functools

import jax
import jax.numpy as jnp
from jax import lax
from jax.experimental import pallas as pl
from jax.experimental.pallas import tpu as pltpu
from jax.experimental.pallas import tpu_sc as plsc

N = 10000
D = 128
DH = 64
DG = 80
HID = 256
E = 320000
NPAD = 10240
NC, NS, L = 2, 16, 16
NW = NC * NS
CHUNK = 80
NCH1 = E // NW // CHUNK
NCH2 = E // NS // CHUNK
RPT = NPAD // NS
ZR = 128

_SC_PARAMS = pltpu.CompilerParams(use_tc_tiling_on_sc=False)


def _mesh():
    return plsc.VectorSubcoreMesh(core_axis_name="c", subcore_axis_name="s")


@functools.partial(
    pl.kernel,
    out_type=jax.ShapeDtypeStruct((NC, 1, NPAD), jnp.float32),
    mesh=_mesh(),
    scratch_types=[
        pltpu.VMEM((NCH1, CHUNK), jnp.int32),
        pltpu.VMEM((CHUNK,), jnp.float32),
        pltpu.VMEM((RPT,), jnp.float32),
        pltpu.VMEM_SHARED((NPAD,), jnp.float32),
        pltpu.SemaphoreType.DMA,
    ],
    compiler_params=_SC_PARAMS,
)
def _deg_kernel(dst_hbm, out_hbm, dstv, ones_v, zbuf, acc, sem):
    cid = lax.axis_index("c")
    sid = lax.axis_index("s")
    w = cid * NS + sid
    zeros = jnp.zeros((L,), jnp.float32)
    ones = jnp.ones((L,), jnp.float32)
    for u in range(CHUNK // L):
        ones_v[pl.ds(u * L, L)] = ones

    @pl.loop(0, RPT // L)
    def _zz(u):
        zbuf[pl.ds(u * L, L)] = zeros

    pltpu.sync_copy(zbuf, acc.at[pl.ds(sid * RPT, RPT)])
    pltpu.sync_copy(dst_hbm.at[w], dstv)
    plsc.subcore_barrier()

    @pl.loop(0, NCH1, unroll=4)
    def _chunk(j):
        pltpu.async_copy(ones_v, acc.at[dstv.at[j]], sem, add=True)

    @pl.loop(0, NCH1, unroll=4)
    def _drain(j):
        pltpu.make_async_copy(ones_v, acc.at[dstv.at[j]], sem).wait()

    plsc.subcore_barrier()

    @pl.when(sid == 0)
    def _out():
        pltpu.sync_copy(acc, out_hbm.at[cid, 0])


def _make_pass(dw):
    @functools.partial(
        pl.kernel,
        out_type=jax.ShapeDtypeStruct((NC * NPAD, dw), jnp.float32),
        mesh=_mesh(),
        scratch_types=[
            pltpu.VMEM((NCH2, CHUNK), jnp.int32),
            pltpu.VMEM((NCH2, CHUNK), jnp.int32),
            pltpu.VMEM((4 * CHUNK, dw), jnp.float32),
            pltpu.VMEM((ZR, dw), jnp.float32),
            pltpu.VMEM_SHARED((NPAD, dw), jnp.float32),
            pltpu.SemaphoreType.DMA,
            pltpu.SemaphoreType.DMA,
        ],
        compiler_params=_SC_PARAMS,
    )
    def pass_kernel(table, srcp, dstp, agg_hbm, srcv, dstv, rows_v, zbuf,
                    acc, sem, ssem):
        cid = lax.axis_index("c")
        sid = lax.axis_index("s")
        zeros = jnp.zeros((L,), jnp.float32)
        tab = table

        @pl.loop(0, ZR)
        def _z(i):
            for u in range(dw // L):
                zbuf[i, pl.ds(u * L, L)] = zeros

        for k in range(RPT // ZR):
            pltpu.sync_copy(zbuf, acc.at[pl.ds(sid * RPT + k * ZR, ZR)])
        pltpu.sync_copy(srcp.at[cid, sid], srcv)
        pltpu.sync_copy(dstp.at[sid], dstv)
        plsc.subcore_barrier()

        pltpu.async_copy(tab.at[srcv.at[0]], rows_v.at[pl.ds(0, CHUNK)],
                         sem)
        pltpu.async_copy(tab.at[srcv.at[1]],
                         rows_v.at[pl.ds(CHUNK, CHUNK)], sem)

        @pl.loop(0, NCH2, unroll=2)
        def _chunk(j):
            cur = lax.bitwise_and(j, 3) * CHUNK
            nxt = lax.bitwise_and(j + 2, 3) * CHUNK

            @pl.when(j >= 2)
            def _drain():
                pltpu.make_async_copy(rows_v.at[pl.ds(nxt, CHUNK)],
                                      acc.at[dstv.at[j - 2]], ssem).wait()

            @pl.when(j < NCH2 - 2)
            def _prefetch():
                pltpu.async_copy(tab.at[srcv.at[j + 2]],
                                 rows_v.at[pl.ds(nxt, CHUNK)], sem)

            pltpu.make_async_copy(tab.at[srcv.at[j]],
                                  rows_v.at[pl.ds(cur, CHUNK)], sem).wait()
            pltpu.async_copy(rows_v.at[pl.ds(cur, CHUNK)],
                             acc.at[dstv.at[j]], ssem, add=True)

        pltpu.make_async_copy(rows_v.at[pl.ds(2 * CHUNK, CHUNK)],
                              acc.at[dstv.at[NCH2 - 2]], ssem).wait()
        pltpu.make_async_copy(rows_v.at[pl.ds(3 * CHUNK, CHUNK)],
                              acc.at[dstv.at[NCH2 - 1]], ssem).wait()
        plsc.subcore_barrier()
        pltpu.sync_copy(acc.at[pl.ds(sid * RPT, RPT)],
                        agg_hbm.at[pl.ds(cid * NPAD + sid * RPT, RPT)])

    return pass_kernel


_pass1 = _make_pass(DG)
_pass2 = _make_pass(DH)


def _rsqrt_body(dp, r):
    r[...] = lax.rsqrt(dp[0, 0] + dp[1, 0] + 1.0)


_rsqrt_call = pl.pallas_call(
    _rsqrt_body,
    out_shape=jax.ShapeDtypeStruct((NPAD // D, D), jnp.float32),
)

_GRID = 10
_BR = NPAD // _GRID


def _scale_body(x, r, o):
    h = pl.program_id(0) // _GRID
    rb = r[...]
    xb = x[...]
    xh = jnp.where(h == 0, xb[:, :DH], xb[:, DH:])
    lane = jax.lax.broadcasted_iota(jnp.int32, (_BR, DG - DH), 1)
    ghost = jnp.where(lane == 0, rb, 0.0)
    o[...] = jnp.concatenate([xh * rb, ghost], axis=1)


_scale_call = pl.pallas_call(
    _scale_body,
    grid=(2 * _GRID,),
    in_specs=[
        pl.BlockSpec((_BR, D), lambda i: (i % _GRID, 0)),
        pl.BlockSpec((_BR, 1), lambda i: (i % _GRID, 0)),
    ],
    out_specs=pl.BlockSpec((_BR, DG), lambda i: (i, 0)),
    out_shape=jax.ShapeDtypeStruct((NC * NPAD, DG), jnp.float32),
)


def _mid_body(ap, t1, r, o):
    rb = r[...]
    rr = rb * rb
    o[...] = (ap[:, :DH] + t1[:, :DH]) * rr


_mid_call = pl.pallas_call(
    _mid_body,
    grid=(2 * _GRID,),
    in_specs=[
        pl.BlockSpec((_BR, DG), lambda i: (i, 0)),
        pl.BlockSpec((_BR, DG), lambda i: (i, 0)),
        pl.BlockSpec((_BR, 1), lambda i: (i % _GRID, 0)),
    ],
    out_specs=pl.BlockSpec((_BR, DH), lambda i: (i, 0)),
    out_shape=jax.ShapeDtypeStruct((NC * NPAD, DH), jnp.float32),
)


def _fin_body(apa, apb, t2a, t2b, ap1, r, w1, w2, b1, b2, o):
    rb = r[...]
    ya = (apa[...] + t2a[...]) * rb
    yb = (apb[...] + t2b[...]) * rb
    y2 = jnp.concatenate([ya, yb], axis=1)
    wc = jnp.dot(w1[...], w2[...], preferred_element_type=jnp.float32)
    bv = jnp.dot(b1[...], w2[...], preferred_element_type=jnp.float32)
    c = (ap1[:, DH:DH + 1] + rb) * rb
    z = jnp.dot(y2, wc, preferred_element_type=jnp.float32) + c * bv + b2[...]
    o[...] = jax.nn.sigmoid(z)


_fin_call = pl.pallas_call(
    _fin_body,
    grid=(_GRID,),
    in_specs=[
        pl.BlockSpec((_BR, DH), lambda i: (i, 0)),
        pl.BlockSpec((_BR, DH), lambda i: (_GRID + i, 0)),
        pl.BlockSpec((_BR, DH), lambda i: (i, 0)),
        pl.BlockSpec((_BR, DH), lambda i: (_GRID + i, 0)),
        pl.BlockSpec((_BR, DG), lambda i: (i, 0)),
        pl.BlockSpec((_BR, 1), lambda i: (i, 0)),
        pl.BlockSpec((D, HID), lambda i: (0, 0)),
        pl.BlockSpec((HID, D), lambda i: (0, 0)),
        pl.BlockSpec((1, HID), lambda i: (0, 0)),
        pl.BlockSpec((1, D), lambda i: (0, 0)),
    ],
    out_specs=pl.BlockSpec((_BR, D), lambda i: (i, 0)),
    out_shape=jax.ShapeDtypeStruct((NPAD, D), jnp.float32),
)


def kernel(x, edge_index, W1, b1, W2, b2):
    src = edge_index[0].astype(jnp.int32)
    dst = edge_index[1].astype(jnp.int32)
    dst32 = dst.reshape(NW, NCH1, CHUNK)
    src16 = src.reshape(NS, NCH2, CHUNK)
    srcp = jnp.stack([src16, src16 + NPAD])
    dstp = dst.reshape(NS, NCH2, CHUNK)
    xp = jnp.pad(x, ((0, NPAD - N), (0, 0)))

    degp = _deg_kernel(dst32)
    r = _rsqrt_call(degp.reshape(NC, 1, NPAD // D, D))
    r2d = r.reshape(NPAD, 1)

    t1 = _scale_call(xp, r2d)
    agg1 = _pass1(t1, srcp, dstp)
    t2 = _mid_call(agg1, t1, r2d)
    agg2 = _pass2(t2, srcp, dstp)

    out = _fin_call(agg2, agg2, t2, t2, agg1, r2d, W1, W2,
                    b1.reshape(1, HID), b2.reshape(1, D))
    return out[:N]
